# baseline (speedup 1.0000x reference)
import functools

import jax
import jax.numpy as jnp
from jax import lax
from jax.experimental import pallas as pl
from jax.experimental.pallas import tpu as pltpu
from jax.experimental.pallas import tpu_sc as plsc

NT = 16
NC = 2
CH = 80
PH = 64
CH_DEG = 40
DH = 128
RB = 400


def _npad(n):
    return -(-n // (NT * 128)) * (NT * 128)


def _fill_zero(ref, rows, width):
    def frow(r, _):
        def lane(l, _):
            ref[r, pl.ds(l * 16, 16)] = jnp.zeros((16,), jnp.float32)
            return 0
        lax.fori_loop(0, width // 16, lane, 0)
        return 0
    lax.fori_loop(0, rows, frow, 0)


def _zero_my_slice(acc, zb_v, t, rows_pt):
    zrows = zb_v.shape[0]
    _fill_zero(zb_v, zrows, zb_v.shape[1])
    for q in range(rows_pt // zrows):
        pltpu.sync_copy(zb_v, acc.at[pl.ds(t * rows_pt + q * zrows, zrows)])


def _sc_degree_body(npad, nchunk, didx_hbm, deg_hbm, didx_v, ones_v, zb_v, acc):
    c = lax.axis_index("core")
    t = lax.axis_index("subcore")
    w = c * NT + t
    rows_pt = npad // NT

    def fill_ones(r, _):
        def lane(l, _):
            ones_v[r, pl.ds(l * 16, 16)] = jnp.ones((16,), jnp.float32)
            return 0
        lax.fori_loop(0, DH // 16, lane, 0)
        return 0
    lax.fori_loop(0, CH_DEG, fill_ones, 0)

    _zero_my_slice(acc, zb_v, t, rows_pt)
    plsc.subcore_barrier()

    pltpu.sync_copy(didx_hbm.at[w], didx_v)

    def step(j, _):
        pltpu.sync_copy(ones_v, acc.at[didx_v.at[j]], add=True)
        return 0
    lax.fori_loop(0, nchunk, step, 0)
    plsc.subcore_barrier()

    pltpu.sync_copy(acc.at[pl.ds(t * rows_pt, rows_pt)],
                    deg_hbm.at[pl.ds(c * npad + t * rows_pt, rows_pt)])


def _sc_scatter_body(npad, nchunk, z2_hbm, sidx2_hbm, didx_hbm, u_hbm,
                     sidx_v, didx_v, rows0_v, rows1_v, sem0, ssem0, ssem1, acc):
    c = lax.axis_index("core")
    t = lax.axis_index("subcore")
    w = c * NT + t
    rows_pt = npad // NT

    _zero_my_slice(acc, rows0_v, t, rows_pt)
    plsc.subcore_barrier()

    def step(p, _):
        j = p * 2
        pltpu.async_copy(z2_hbm.at[sidx_v.at[j]], rows0_v, sem0).wait()
        s0 = pltpu.async_copy(rows0_v, acc.at[didx_v.at[j]], ssem0, add=True)
        pltpu.async_copy(z2_hbm.at[sidx_v.at[j + 1]], rows1_v, sem0).wait()
        s1 = pltpu.async_copy(rows1_v, acc.at[didx_v.at[j + 1]], ssem1, add=True)
        s0.wait()
        s1.wait()
        return 0

    for phase in range(nchunk // PH):
        pltpu.sync_copy(sidx2_hbm.at[w, pl.ds(phase * PH, PH)], sidx_v)
        pltpu.sync_copy(didx_hbm.at[t, pl.ds(phase * PH, PH)], didx_v)
        lax.fori_loop(0, PH // 2, step, 0)
    plsc.subcore_barrier()

    pltpu.sync_copy(acc.at[pl.ds(t * rows_pt, rows_pt)],
                    u_hbm.at[pl.ds(c * npad + t * rows_pt, rows_pt)])


def _sc_mesh():
    return plsc.VectorSubcoreMesh(core_axis_name="core", subcore_axis_name="subcore",
                                  num_cores=NC, num_subcores=NT)


def _sc_degree(didx_deg, n):
    npad = _npad(n)
    nchunk = didx_deg.shape[1]
    deg2 = pl.kernel(
        functools.partial(_sc_degree_body, npad, nchunk),
        out_type=jax.ShapeDtypeStruct((NC * npad, DH), jnp.float32),
        mesh=_sc_mesh(),
        scratch_types=[
            pltpu.VMEM((nchunk, CH_DEG), jnp.int32),
            pltpu.VMEM((CH_DEG, DH), jnp.float32),
            pltpu.VMEM((80, DH), jnp.float32),
            pltpu.VMEM_SHARED((npad, DH), jnp.float32),
        ],
    )(didx_deg)
    return deg2.reshape(NC, npad, DH)


def _sc_scatter(z2, sidx2, didx, n):
    npad = _npad(n)
    nchunk = didx.shape[1]
    u2 = pl.kernel(
        functools.partial(_sc_scatter_body, npad, nchunk),
        out_type=jax.ShapeDtypeStruct((NC * npad, DH), jnp.float32),
        mesh=_sc_mesh(),
        scratch_types=[
            pltpu.VMEM((PH, CH), jnp.int32),
            pltpu.VMEM((PH, CH), jnp.int32),
            pltpu.VMEM((CH, DH), jnp.float32),
            pltpu.VMEM((CH, DH), jnp.float32),
            pltpu.SemaphoreType.DMA,
            pltpu.SemaphoreType.DMA,
            pltpu.SemaphoreType.DMA,
            pltpu.VMEM_SHARED((npad, DH), jnp.float32),
        ],
    )(z2, sidx2, didx)
    return u2.reshape(NC, npad, DH)


def _dinv_from_deg(deg_ref):
    degsum = deg_ref[0, :, 0:1] + deg_ref[1, :, 0:1] + 1.0
    return lax.rsqrt(degsum)


def _enc_body(x_ref, pe_ref, pw1_ref, pb1_ref, pw2_ref, pb2_ref,
              mw1a_ref, mw1b_ref, mb1_ref, mw2_ref, mb2_ref,
              w0_ref, deg_ref, h_ref, z_ref):
    f32 = jnp.float32
    pe = pe_ref[...]
    t1 = jnp.maximum(jnp.dot(pe, pw1_ref[...], preferred_element_type=f32)
                     + pb1_ref[...], 0.0)
    pe2 = jnp.maximum(jnp.dot(t1, pw2_ref[...], preferred_element_type=f32)
                      + pb2_ref[...], 0.0)
    t2 = jnp.maximum(jnp.dot(x_ref[...], mw1a_ref[...], preferred_element_type=f32)
                     + jnp.dot(pe2, mw1b_ref[...], preferred_element_type=f32)
                     + mb1_ref[...], 0.0)
    h = jnp.dot(t2, mw2_ref[...], preferred_element_type=f32) + mb2_ref[...]
    h_ref[...] = h
    dinv = _dinv_from_deg(deg_ref)
    z = dinv * jnp.dot(h, w0_ref[...], preferred_element_type=f32)
    z_ref[0] = z[:, :DH]
    z_ref[1] = z[:, DH:]


def _layer_body(h_ref, z_ref, u_ref, deg_ref, b_ref, w_ref, hout_ref, zout_ref):
    f32 = jnp.float32
    dinv = _dinv_from_deg(deg_ref)
    agg = jnp.concatenate([u_ref[0] + z_ref[0], u_ref[1] + z_ref[1]], axis=1)
    h = h_ref[...] + dinv * agg + b_ref[...]
    hout_ref[...] = h
    z = dinv * jnp.dot(h, w_ref[...], preferred_element_type=f32)
    zout_ref[0] = z[:, :DH]
    zout_ref[1] = z[:, DH:]


def _final_body(nrb, h_ref, z_ref, u_ref, deg_ref, b_ref, batch_ref,
                pw_ref, pb_ref, s_ref, c_ref, out_ref):
    f32 = jnp.float32
    i = pl.program_id(0)
    dinv = _dinv_from_deg(deg_ref)
    agg = jnp.concatenate([u_ref[0] + z_ref[0], u_ref[1] + z_ref[1]], axis=1)
    h = h_ref[...] + dinv * agg + b_ref[...]
    nrm = jnp.sqrt(jnp.sum(h * h, axis=1, keepdims=True))
    hn = h / jnp.maximum(nrm, 1e-12)
    iota = lax.broadcasted_iota(jnp.int32, (h.shape[0], 64), 1)
    p = (batch_ref[...] == iota).astype(f32)
    s_part = lax.dot_general(p, hn, (((0,), (0,)), ((), ())),
                             precision=lax.Precision.HIGHEST,
                             preferred_element_type=f32)
    ones8 = jnp.ones((h.shape[0], 8), f32)
    c_part = lax.dot_general(p, ones8, (((0,), (0,)), ((), ())),
                             precision=lax.Precision.HIGHEST,
                             preferred_element_type=f32)

    @pl.when(i == 0)
    def _():
        s_ref[...] = s_part
        c_ref[...] = c_part

    @pl.when(i > 0)
    def _():
        s_ref[...] += s_part
        c_ref[...] += c_part

    @pl.when(i == nrb - 1)
    def _():
        pooled = s_ref[...] / jnp.maximum(c_ref[...][:, 0:1], 1.0)
        out_ref[...] = (jnp.dot(pooled, pw_ref[...], preferred_element_type=f32)
                        + pb_ref[...])


def _full(shape):
    return pl.BlockSpec(shape, lambda i: tuple(0 for _ in shape))


def _rows(width):
    return pl.BlockSpec((RB, width), lambda i: (i, 0))


def _halves():
    return pl.BlockSpec((2, RB, DH), lambda i: (0, i, 0))


def _deg_spec():
    return pl.BlockSpec((2, RB, DH), lambda i: (0, i, 0))


def _tc_encoder(x, pe, pw1, pb1, pw2, pb2, mw1a, mw1b, mb1, mw2, mb2, w0, deg2, n):
    nrb = n // RB
    dim_x = x.shape[1]
    h, z = pl.pallas_call(
        _enc_body,
        grid=(nrb,),
        in_specs=[
            _rows(dim_x), _rows(pe.shape[1]),
            _full(pw1.shape), _full(pb1.shape), _full(pw2.shape), _full(pb2.shape),
            _full(mw1a.shape), _full(mw1b.shape), _full(mb1.shape),
            _full(mw2.shape), _full(mb2.shape), _full(w0.shape),
            _deg_spec(),
        ],
        out_specs=[_rows(2 * DH), _halves()],
        out_shape=[jax.ShapeDtypeStruct((n, 2 * DH), jnp.float32),
                   jax.ShapeDtypeStruct((2, n, DH), jnp.float32)],
    )(x, pe, pw1, pb1, pw2, pb2, mw1a, mw1b, mb1, mw2, mb2, w0, deg2)
    return h, z


def _tc_layer(h, z, u, deg2, b, w, n):
    nrb = n // RB
    hout, zout = pl.pallas_call(
        _layer_body,
        grid=(nrb,),
        in_specs=[_rows(2 * DH), _halves(), _halves(), _deg_spec(),
                  _full(b.shape), _full(w.shape)],
        out_specs=[_rows(2 * DH), _halves()],
        out_shape=[jax.ShapeDtypeStruct((n, 2 * DH), jnp.float32),
                   jax.ShapeDtypeStruct((2, n, DH), jnp.float32)],
    )(h, z, u, deg2, b, w)
    return hout, zout


def _tc_final(h, z, u, deg2, b, batch2, pw, pb, n):
    nrb = n // RB
    _, _, out = pl.pallas_call(
        functools.partial(_final_body, nrb),
        grid=(nrb,),
        in_specs=[_rows(2 * DH), _halves(), _halves(), _deg_spec(),
                  _full(b.shape), _rows(1), _full(pw.shape), _full(pb.shape)],
        out_specs=[_full((64, 2 * DH)), _full((64, 8)), _full((64, 1))],
        out_shape=[jax.ShapeDtypeStruct((64, 2 * DH), jnp.float32),
                   jax.ShapeDtypeStruct((64, 8), jnp.float32),
                   jax.ShapeDtypeStruct((64, 1), jnp.float32)],
        compiler_params=pltpu.CompilerParams(
            dimension_semantics=("arbitrary",)),
    )(h, z, u, deg2, b, batch2, pw, pb)
    return out


def kernel(x, pestat_GPSE, bn_w, bn_b, bn_mean, bn_var,
           pe_w1, pe_b1, pe_w2, pe_b2,
           mlp_w1, mlp_b1, mlp_w2, mlp_b2,
           gcn_w, gcn_b, post_w, post_b,
           edge_index, batch):
    n = x.shape[0]
    e = edge_index.shape[1]
    num_layers = gcn_w.shape[0]
    dim_x = x.shape[1]
    ept = -(-e // NT)
    nchunk_min = -(-ept // CH)
    nchunk = -(-nchunk_min // PH) * PH

    scale = bn_w * lax.rsqrt(bn_var + 1e-5)
    shift = bn_b - bn_mean * scale
    pw1 = scale[:, None] * pe_w1
    pb1 = (pe_b1 + shift @ pe_w1)[None, :]
    pb2 = pe_b2[None, :]
    mw1a = mlp_w1[:dim_x]
    mw1b = mlp_w1[dim_x:]
    mb1 = mlp_b1[None, :]
    mb2 = mlp_b2[None, :]

    npad = _npad(n)
    src = edge_index[0]
    dst = edge_index[1]
    pad = NT * nchunk * CH - e
    src_pad = jnp.concatenate([src, jnp.zeros((pad,), jnp.int32)])
    dst_pad = jnp.concatenate([dst, jnp.full((pad,), npad - 1, jnp.int32)])
    sidx16 = src_pad.reshape(NT, nchunk, CH)
    sidx2 = jnp.concatenate([sidx16, sidx16 + n], axis=0)
    didx16 = dst_pad.reshape(NT, nchunk, CH)
    didx_deg = dst.reshape(NC * NT, (e // (NC * NT)) // CH_DEG, CH_DEG)
    batch2 = batch[:, None]

    deg2 = _sc_degree(didx_deg, n)

    h, z = _tc_encoder(x, pestat_GPSE, pw1, pb1, pe_w2, pb2,
                       mw1a, mw1b, mb1, mlp_w2, mb2, gcn_w[0], deg2, n)
    for i in range(num_layers):
        u = _sc_scatter(z.reshape(NC * n, DH), sidx2, didx16, n)
        if i + 1 < num_layers:
            h, z = _tc_layer(h, z, u, deg2, gcn_b[i][None, :], gcn_w[i + 1], n)
    out = _tc_final(h, z, u, deg2, gcn_b[num_layers - 1][None, :],
                    batch2, post_w, post_b[None, :], n)
    return out


# speedup vs baseline: 1.7086x; 1.7086x over previous
import functools

import jax
import jax.numpy as jnp
from jax import lax
from jax.experimental import pallas as pl
from jax.experimental.pallas import tpu as pltpu
from jax.experimental.pallas import tpu_sc as plsc

NT = 16
NC = 2
CH = 80
CH_DEG = 40
DH = 128
RB = 400


def _npad(n):
    return -(-n // (NT * 128)) * (NT * 128)


def _fill_zero(ref, rows, width):
    def frow(r, _):
        def lane(l, _):
            ref[r, pl.ds(l * 16, 16)] = jnp.zeros((16,), jnp.float32)
            return 0
        lax.fori_loop(0, width // 16, lane, 0)
        return 0
    lax.fori_loop(0, rows, frow, 0)


def _zero_my_slice(acc, zb_v, t, rows_pt):
    zrows = zb_v.shape[0]
    _fill_zero(zb_v, zrows, zb_v.shape[1])
    for q in range(rows_pt // zrows):
        pltpu.sync_copy(zb_v, acc.at[pl.ds(t * rows_pt + q * zrows, zrows)])


DW = 16


def _sc_degree_body(npad, nchunk, didx_hbm, deg_hbm, didx_v, ones_v, zb_v, acc):
    c = lax.axis_index("core")
    t = lax.axis_index("subcore")
    w = c * NT + t
    rows_pt = npad // NT

    def fill_ones(r, _):
        ones_v[r, :] = jnp.ones((16,), jnp.float32)
        return 0
    lax.fori_loop(0, CH_DEG, fill_ones, 0)

    _zero_my_slice(acc, zb_v, t, rows_pt)
    plsc.subcore_barrier()

    pltpu.sync_copy(didx_hbm.at[w], didx_v)

    def step(j, _):
        pltpu.sync_copy(ones_v, acc.at[didx_v.at[j]], add=True)
        return 0
    lax.fori_loop(0, nchunk, step, 0)
    plsc.subcore_barrier()

    pltpu.sync_copy(acc.at[pl.ds(t * rows_pt, rows_pt)],
                    deg_hbm.at[pl.ds(c * npad + t * rows_pt, rows_pt)])


def _sc_scatter_body(npad, nchunk, z2_hbm, sidx2_hbm, didx_hbm, u_hbm,
                     sidx_v, didx_v, rows0_v, sem0, acc):
    c = lax.axis_index("core")
    t = lax.axis_index("subcore")
    w = c * NT + t
    rows_pt = npad // NT

    _zero_my_slice(acc, rows0_v, t, rows_pt)
    plsc.subcore_barrier()

    pltpu.sync_copy(sidx2_hbm.at[w], sidx_v)
    pltpu.sync_copy(didx_hbm.at[t], didx_v)

    def step(j, _):
        pltpu.async_copy(z2_hbm.at[sidx_v.at[j]], rows0_v, sem0).wait()
        pltpu.sync_copy(rows0_v, acc.at[didx_v.at[j]], add=True)
        return 0
    lax.fori_loop(0, nchunk, step, 0)
    plsc.subcore_barrier()

    pltpu.sync_copy(acc.at[pl.ds(t * rows_pt, rows_pt)],
                    u_hbm.at[pl.ds(c * npad + t * rows_pt, rows_pt)])


def _sc_mesh():
    return plsc.VectorSubcoreMesh(core_axis_name="core", subcore_axis_name="subcore",
                                  num_cores=NC, num_subcores=NT)


def _sc_degree(didx_deg, n):
    npad = _npad(n)
    nchunk = didx_deg.shape[1]
    deg2 = pl.kernel(
        functools.partial(_sc_degree_body, npad, nchunk),
        out_type=jax.ShapeDtypeStruct((NC * npad, DW), jnp.float32),
        mesh=_sc_mesh(),
        scratch_types=[
            pltpu.VMEM((nchunk, CH_DEG), jnp.int32),
            pltpu.VMEM((CH_DEG, DW), jnp.float32),
            pltpu.VMEM((80, DW), jnp.float32),
            pltpu.VMEM_SHARED((npad, DW), jnp.float32),
        ],
    )(didx_deg)
    return deg2.reshape(NC, npad, DW)


def _sc_scatter(z2, sidx2, didx, n):
    npad = _npad(n)
    nchunk = didx.shape[1]
    u2 = pl.kernel(
        functools.partial(_sc_scatter_body, npad, nchunk),
        out_type=jax.ShapeDtypeStruct((NC * npad, DH), jnp.float32),
        mesh=_sc_mesh(),
        scratch_types=[
            pltpu.VMEM((nchunk, CH), jnp.int32),
            pltpu.VMEM((nchunk, CH), jnp.int32),
            pltpu.VMEM((CH, DH), jnp.float32),
            pltpu.SemaphoreType.DMA,
            pltpu.VMEM_SHARED((npad, DH), jnp.float32),
        ],
    )(z2, sidx2, didx)
    return u2.reshape(NC, npad, DH)


def _dinv_from_deg(deg_ref):
    degsum = deg_ref[0, :, 0:1] + deg_ref[1, :, 0:1] + 1.0
    return lax.rsqrt(degsum)


def _enc_body(x_ref, pe_ref, pw1_ref, pb1_ref, pw2_ref, pb2_ref,
              mw1a_ref, mw1b_ref, mb1_ref, mw2_ref, mb2_ref,
              w0_ref, deg_ref, h_ref, z_ref):
    f32 = jnp.float32
    pe = pe_ref[...]
    t1 = jnp.maximum(jnp.dot(pe, pw1_ref[...], preferred_element_type=f32)
                     + pb1_ref[...], 0.0)
    pe2 = jnp.maximum(jnp.dot(t1, pw2_ref[...], preferred_element_type=f32)
                      + pb2_ref[...], 0.0)
    t2 = jnp.maximum(jnp.dot(x_ref[...], mw1a_ref[...], preferred_element_type=f32)
                     + jnp.dot(pe2, mw1b_ref[...], preferred_element_type=f32)
                     + mb1_ref[...], 0.0)
    h = jnp.dot(t2, mw2_ref[...], preferred_element_type=f32) + mb2_ref[...]
    h_ref[...] = h
    dinv = _dinv_from_deg(deg_ref)
    z = dinv * jnp.dot(h, w0_ref[...], preferred_element_type=f32)
    z_ref[0] = z[:, :DH]
    z_ref[1] = z[:, DH:]


def _layer_body(h_ref, z_ref, u_ref, deg_ref, b_ref, w_ref, hout_ref, zout_ref):
    f32 = jnp.float32
    dinv = _dinv_from_deg(deg_ref)
    agg = jnp.concatenate([u_ref[0] + z_ref[0], u_ref[1] + z_ref[1]], axis=1)
    h = h_ref[...] + dinv * agg + b_ref[...]
    hout_ref[...] = h
    z = dinv * jnp.dot(h, w_ref[...], preferred_element_type=f32)
    zout_ref[0] = z[:, :DH]
    zout_ref[1] = z[:, DH:]


def _final_body(nrb, h_ref, z_ref, u_ref, deg_ref, b_ref, batch_ref,
                pw_ref, pb_ref, s_ref, c_ref, out_ref):
    f32 = jnp.float32
    i = pl.program_id(0)
    dinv = _dinv_from_deg(deg_ref)
    agg = jnp.concatenate([u_ref[0] + z_ref[0], u_ref[1] + z_ref[1]], axis=1)
    h = h_ref[...] + dinv * agg + b_ref[...]
    nrm = jnp.sqrt(jnp.sum(h * h, axis=1, keepdims=True))
    hn = h / jnp.maximum(nrm, 1e-12)
    iota = lax.broadcasted_iota(jnp.int32, (h.shape[0], 64), 1)
    p = (batch_ref[...] == iota).astype(f32)
    s_part = lax.dot_general(p, hn, (((0,), (0,)), ((), ())),
                             precision=lax.Precision.HIGHEST,
                             preferred_element_type=f32)
    ones8 = jnp.ones((h.shape[0], 8), f32)
    c_part = lax.dot_general(p, ones8, (((0,), (0,)), ((), ())),
                             precision=lax.Precision.HIGHEST,
                             preferred_element_type=f32)

    @pl.when(i == 0)
    def _():
        s_ref[...] = s_part
        c_ref[...] = c_part

    @pl.when(i > 0)
    def _():
        s_ref[...] += s_part
        c_ref[...] += c_part

    @pl.when(i == nrb - 1)
    def _():
        pooled = s_ref[...] / jnp.maximum(c_ref[...][:, 0:1], 1.0)
        out_ref[...] = (jnp.dot(pooled, pw_ref[...], preferred_element_type=f32)
                        + pb_ref[...])


def _full(shape):
    return pl.BlockSpec(shape, lambda i: tuple(0 for _ in shape))


def _rows(width):
    return pl.BlockSpec((RB, width), lambda i: (i, 0))


def _halves():
    return pl.BlockSpec((2, RB, DH), lambda i: (0, i, 0))


def _deg_spec():
    return pl.BlockSpec((2, RB, DW), lambda i: (0, i, 0))


def _tc_encoder(x, pe, pw1, pb1, pw2, pb2, mw1a, mw1b, mb1, mw2, mb2, w0, deg2, n):
    nrb = n // RB
    dim_x = x.shape[1]
    h, z = pl.pallas_call(
        _enc_body,
        grid=(nrb,),
        in_specs=[
            _rows(dim_x), _rows(pe.shape[1]),
            _full(pw1.shape), _full(pb1.shape), _full(pw2.shape), _full(pb2.shape),
            _full(mw1a.shape), _full(mw1b.shape), _full(mb1.shape),
            _full(mw2.shape), _full(mb2.shape), _full(w0.shape),
            _deg_spec(),
        ],
        out_specs=[_rows(2 * DH), _halves()],
        out_shape=[jax.ShapeDtypeStruct((n, 2 * DH), jnp.float32),
                   jax.ShapeDtypeStruct((2, n, DH), jnp.float32)],
    )(x, pe, pw1, pb1, pw2, pb2, mw1a, mw1b, mb1, mw2, mb2, w0, deg2)
    return h, z


def _tc_layer(h, z, u, deg2, b, w, n):
    nrb = n // RB
    hout, zout = pl.pallas_call(
        _layer_body,
        grid=(nrb,),
        in_specs=[_rows(2 * DH), _halves(), _halves(), _deg_spec(),
                  _full(b.shape), _full(w.shape)],
        out_specs=[_rows(2 * DH), _halves()],
        out_shape=[jax.ShapeDtypeStruct((n, 2 * DH), jnp.float32),
                   jax.ShapeDtypeStruct((2, n, DH), jnp.float32)],
    )(h, z, u, deg2, b, w)
    return hout, zout


def _tc_final(h, z, u, deg2, b, batch2, pw, pb, n):
    nrb = n // RB
    _, _, out = pl.pallas_call(
        functools.partial(_final_body, nrb),
        grid=(nrb,),
        in_specs=[_rows(2 * DH), _halves(), _halves(), _deg_spec(),
                  _full(b.shape), _rows(1), _full(pw.shape), _full(pb.shape)],
        out_specs=[_full((64, 2 * DH)), _full((64, 8)), _full((64, 1))],
        out_shape=[jax.ShapeDtypeStruct((64, 2 * DH), jnp.float32),
                   jax.ShapeDtypeStruct((64, 8), jnp.float32),
                   jax.ShapeDtypeStruct((64, 1), jnp.float32)],
        compiler_params=pltpu.CompilerParams(
            dimension_semantics=("arbitrary",)),
    )(h, z, u, deg2, b, batch2, pw, pb)
    return out


def kernel(x, pestat_GPSE, bn_w, bn_b, bn_mean, bn_var,
           pe_w1, pe_b1, pe_w2, pe_b2,
           mlp_w1, mlp_b1, mlp_w2, mlp_b2,
           gcn_w, gcn_b, post_w, post_b,
           edge_index, batch):
    n = x.shape[0]
    e = edge_index.shape[1]
    num_layers = gcn_w.shape[0]
    dim_x = x.shape[1]
    ept = -(-e // NT)
    nchunk = -(-ept // CH)

    scale = bn_w * lax.rsqrt(bn_var + 1e-5)
    shift = bn_b - bn_mean * scale
    pw1 = scale[:, None] * pe_w1
    pb1 = (pe_b1 + shift @ pe_w1)[None, :]
    pb2 = pe_b2[None, :]
    mw1a = mlp_w1[:dim_x]
    mw1b = mlp_w1[dim_x:]
    mb1 = mlp_b1[None, :]
    mb2 = mlp_b2[None, :]

    npad = _npad(n)
    src = edge_index[0]
    dst = edge_index[1]
    pad = NT * nchunk * CH - e
    src_pad = jnp.concatenate([src, jnp.zeros((pad,), jnp.int32)])
    dst_pad = jnp.concatenate([dst, jnp.full((pad,), npad - 1, jnp.int32)])
    sidx16 = src_pad.reshape(NT, nchunk, CH)
    sidx2 = jnp.concatenate([sidx16, sidx16 + n], axis=0)
    didx16 = dst_pad.reshape(NT, nchunk, CH)
    didx_deg = dst.reshape(NC * NT, (e // (NC * NT)) // CH_DEG, CH_DEG)
    batch2 = batch[:, None]

    deg2 = _sc_degree(didx_deg, n)

    h, z = _tc_encoder(x, pestat_GPSE, pw1, pb1, pe_w2, pb2,
                       mw1a, mw1b, mb1, mlp_w2, mb2, gcn_w[0], deg2, n)
    for i in range(num_layers):
        u = _sc_scatter(z.reshape(NC * n, DH), sidx2, didx16, n)
        if i + 1 < num_layers:
            h, z = _tc_layer(h, z, u, deg2, gcn_b[i][None, :], gcn_w[i + 1], n)
    out = _tc_final(h, z, u, deg2, gcn_b[num_layers - 1][None, :],
                    batch2, post_w, post_b[None, :], n)
    return out
